# Initial kernel scaffold; baseline (speedup 1.0000x reference)
import functools

import jax
import jax.numpy as jnp
from jax import lax
from jax.experimental import pallas as pl
from jax.experimental.pallas import tpu as pltpu
from jax.experimental.pallas import tpu_sc as plsc

EMBED_DIM = 64
POS_DIM = 32
OUT_DIM = EMBED_DIM + 2 * POS_DIM
B, L = 4096, 200
N_TOK = B * L

NC, NS = 2, 16
NW = NC * NS
TOK_PER_W = N_TOK // NW
T = 512
STEPS = TOK_PER_W // T
SUB = 128
NSUB = T // SUB


def _emb_body(wid_hbm, p1_hbm, p2_hbm, wtab_hbm, ptab_hbm, out_hbm,
              widx, p1idx, p2idx, wbuf, p1buf, p2buf, sem):
    c = lax.axis_index("c")
    s = lax.axis_index("s")
    wid = s * NC + c
    base0 = wid * TOK_PER_W

    def step(i, carry):
        base = base0 + i * T
        pltpu.sync_copy(wid_hbm.at[pl.ds(base, T)], widx)
        pltpu.sync_copy(p1_hbm.at[pl.ds(base, T)], p1idx)
        pltpu.sync_copy(p2_hbm.at[pl.ds(base, T)], p2idx)
        handles = []
        for j in range(NSUB):
            sl = pl.ds(j * SUB, SUB)
            handles.append(
                pltpu.async_copy(wtab_hbm.at[widx.at[sl]], wbuf.at[sl], sem))
            handles.append(
                pltpu.async_copy(ptab_hbm.at[p1idx.at[sl]], p1buf.at[sl], sem))
            handles.append(
                pltpu.async_copy(ptab_hbm.at[p2idx.at[sl]], p2buf.at[sl], sem))
        for h in handles:
            h.wait()
        row = pl.ds(base, T)
        pltpu.sync_copy(wbuf, out_hbm.at[row, pl.ds(0, EMBED_DIM)])
        pltpu.sync_copy(p1buf, out_hbm.at[row, pl.ds(EMBED_DIM, POS_DIM)])
        pltpu.sync_copy(p2buf, out_hbm.at[row, pl.ds(EMBED_DIM + POS_DIM, POS_DIM)])
        return carry

    lax.fori_loop(0, STEPS, step, 0)


@functools.partial(
    pl.kernel,
    out_type=jax.ShapeDtypeStruct((N_TOK, OUT_DIM), jnp.float32),
    mesh=plsc.VectorSubcoreMesh(core_axis_name="c", subcore_axis_name="s"),
    scratch_types=[
        pltpu.VMEM((T,), jnp.int32),
        pltpu.VMEM((T,), jnp.int32),
        pltpu.VMEM((T,), jnp.int32),
        pltpu.VMEM((T, EMBED_DIM), jnp.float32),
        pltpu.VMEM((T, POS_DIM), jnp.float32),
        pltpu.VMEM((T, POS_DIM), jnp.float32),
        pltpu.SemaphoreType.DMA,
    ],
)
def _emb_kernel(*refs):
    _emb_body(*refs)


def kernel(word_id, pos_1, pos_2, word_table, pos_table):
    out = _emb_kernel(
        word_id.reshape(N_TOK),
        pos_1.reshape(N_TOK),
        pos_2.reshape(N_TOK),
        word_table,
        pos_table,
    )
    return out.reshape(B, L, OUT_DIM)


# speedup vs baseline: 4.7412x; 4.7412x over previous
import functools

import jax
import jax.numpy as jnp
from jax import lax
from jax.experimental import pallas as pl
from jax.experimental.pallas import tpu as pltpu
from jax.experimental.pallas import tpu_sc as plsc

EMBED_DIM = 64
POS_DIM = 32
OUT_DIM = EMBED_DIM + 2 * POS_DIM
B, L = 4096, 200
N_TOK = B * L

NC, NS = 2, 16
NW = NC * NS
TOK_PER_W = N_TOK // NW
T = 512
STEPS = TOK_PER_W // T
SUB = 128
NSUB = T // SUB


def _emb_body(wid_hbm, p1_hbm, p2_hbm, wtab_hbm, ptab_hbm, out_hbm,
              widx, p1idx, p2idx, wbuf, p1buf, p2buf, sem):
    c = lax.axis_index("c")
    s = lax.axis_index("s")
    wid = s * NC + c
    base0 = wid * TOK_PER_W

    def step(i, carry):
        base = base0 + i * T
        pltpu.sync_copy(wid_hbm.at[pl.ds(base, T)], widx)
        pltpu.sync_copy(p1_hbm.at[pl.ds(base, T)], p1idx)
        pltpu.sync_copy(p2_hbm.at[pl.ds(base, T)], p2idx)
        handles = []
        for j in range(NSUB):
            sl = pl.ds(j * SUB, SUB)
            handles.append(
                pltpu.async_copy(wtab_hbm.at[widx.at[sl]], wbuf.at[sl], sem))
            handles.append(
                pltpu.async_copy(ptab_hbm.at[p1idx.at[sl]], p1buf.at[sl], sem))
            handles.append(
                pltpu.async_copy(ptab_hbm.at[p2idx.at[sl]], p2buf.at[sl], sem))
        for h in handles:
            h.wait()
        row = pl.ds(base, T)
        pltpu.sync_copy(wbuf, out_hbm.at[row, pl.ds(0, EMBED_DIM)])
        pltpu.sync_copy(p1buf, out_hbm.at[row, pl.ds(EMBED_DIM, POS_DIM)])
        pltpu.sync_copy(p2buf, out_hbm.at[row, pl.ds(EMBED_DIM + POS_DIM, POS_DIM)])
        return carry

    lax.fori_loop(0, STEPS, step, 0)


@functools.partial(
    pl.kernel,
    out_type=jax.ShapeDtypeStruct((N_TOK, OUT_DIM), jnp.float32),
    mesh=plsc.VectorSubcoreMesh(core_axis_name="c", subcore_axis_name="s"),
    compiler_params=pltpu.CompilerParams(use_tc_tiling_on_sc=False),
    scratch_types=[
        pltpu.VMEM((T,), jnp.int32),
        pltpu.VMEM((T,), jnp.int32),
        pltpu.VMEM((T,), jnp.int32),
        pltpu.VMEM((T, EMBED_DIM), jnp.float32),
        pltpu.VMEM((T, POS_DIM), jnp.float32),
        pltpu.VMEM((T, POS_DIM), jnp.float32),
        pltpu.SemaphoreType.DMA,
    ],
)
def _emb_kernel(*refs):
    _emb_body(*refs)


def kernel(word_id, pos_1, pos_2, word_table, pos_table):
    out = _emb_kernel(
        word_id.reshape(N_TOK),
        pos_1.reshape(N_TOK),
        pos_2.reshape(N_TOK),
        word_table,
        pos_table,
    )
    return out.reshape(B, L, OUT_DIM)
